# baseline (speedup 1.0000x reference)
import functools

import jax
import jax.numpy as jnp
from jax import lax
from jax.experimental import pallas as pl
from jax.experimental.pallas import tpu as pltpu
from jax.experimental.pallas import tpu_sc as plsc

_N = 50000
_F = 128
_S = 9
_O = 128

_MM_BLOCK = 400
_MM_GRID = _N // _MM_BLOCK

_NC = 2
_NS = 16
_NW = _NC * _NS
_CPW = 1568
_CPW_LAST = _N - (_NW - 1) * _CPW
_CH = 8
_ROWS = _CH * _S
_IDXW = _CPW * _S
_NCHUNK = _CPW // _CH
_NCHUNK_LAST = _CPW_LAST // _CH
_LANE = 16
_GROUPS = _O // _LANE


def _mm_body(x_ref, wt_ref, y_ref):
    y_ref[...] = jnp.dot(x_ref[...], wt_ref[...],
                         preferred_element_type=jnp.float32)


def _tc_matmul(x2, wt):
    return pl.pallas_call(
        _mm_body,
        grid=(_MM_GRID, _S),
        in_specs=[
            pl.BlockSpec((_MM_BLOCK, _F), lambda i, s: (i, 0)),
            pl.BlockSpec((_F, _O), lambda i, s: (0, s)),
        ],
        out_specs=pl.BlockSpec((_MM_BLOCK, _O), lambda i, s: (s * _MM_GRID + i, 0)),
        out_shape=jax.ShapeDtypeStruct((_N * _S, _O), jnp.float32),
    )(x2, wt)


@functools.partial(
    pl.kernel,
    out_type=jax.ShapeDtypeStruct((_N, _O), jnp.float32),
    mesh=plsc.VectorSubcoreMesh(core_axis_name="c", subcore_axis_name="s"),
    scratch_types=[
        pltpu.VMEM((_IDXW,), jnp.int32),
        pltpu.VMEM((_ROWS, _O), jnp.float32),
        pltpu.VMEM((_ROWS, _O), jnp.float32),
        pltpu.VMEM((_CH, _O), jnp.float32),
        pltpu.VMEM((_CH, _O), jnp.float32),
        pltpu.VMEM((_O,), jnp.float32),
        pltpu.SemaphoreType.DMA,
        pltpu.SemaphoreType.DMA,
        pltpu.SemaphoreType.DMA,
        pltpu.SemaphoreType.DMA,
    ],
)
def _sc_gather_reduce(y_hbm, idx_hbm, b_hbm, out_hbm,
                      idx_v, rows0, rows1, outb0, outb1, bias_v,
                      sem0, sem1, semo0, semo1):
    wid = lax.axis_index("s") * _NC + lax.axis_index("c")
    node_base = wid * _CPW
    nchunks = jnp.where(wid == _NW - 1, _NCHUNK_LAST, _NCHUNK)

    pltpu.sync_copy(idx_hbm.at[pl.ds(wid * _IDXW, _IDXW)], idx_v)
    pltpu.sync_copy(b_hbm, bias_v)

    def gather(g, rows, sem):
        src = y_hbm.at[idx_v.at[pl.ds(g * _ROWS, _ROWS)]]
        return pltpu.make_async_copy(src, rows, sem)

    gather(0, rows0, sem0).start()
    gather(1, rows1, sem1).start()

    def out_copy(g, outb, semo):
        dst = out_hbm.at[pl.ds(node_base + g * _CH, _CH)]
        return pltpu.make_async_copy(outb, dst, semo)

    def compute(g, rows, outb):
        for n in range(_CH):
            nid = node_base + g * _CH + n
            keep = (nid != _N - 1).astype(jnp.float32)
            for j in range(_GROUPS):
                sl = pl.ds(j * _LANE, _LANE)
                v = rows[n * _S + 0, sl]
                for s in range(1, _S):
                    v = v + rows[n * _S + s, sl]
                v = v + bias_v[sl]
                v = jnp.where(v > 0.0, v, jnp.exp(v) - 1.0)
                outb[n, sl] = v * keep

    def body(h, carry):
        for parity, (rows, sem, outb, semo) in enumerate((
                (rows0, sem0, outb0, semo0), (rows1, sem1, outb1, semo1))):
            g = 2 * h + parity
            gather(g, rows, sem).wait()

            @pl.when(g >= 2)
            def _():
                out_copy(g - 2, outb, semo).wait()

            compute(g, rows, outb)
            out_copy(g, outb, semo).start()
            nxt = g + 2

            @pl.when(nxt < nchunks)
            def _():
                gather(nxt, rows, sem).start()
        return carry

    lax.fori_loop(0, nchunks // 2, body, 0)
    out_copy(nchunks - 2, outb0, semo0).wait()
    out_copy(nchunks - 1, outb1, semo1).wait()


def kernel(x, spiral_adj, W, b):
    B, N, F = x.shape
    S = spiral_adj.shape[-1]
    O = W.shape[0]
    assert (B, N, F, S, O) == (1, _N, _F, _S, _O)

    x2 = x.reshape(N, F)
    wt = jnp.transpose(W.reshape(O, S, F), (2, 1, 0)).reshape(F, S * O)
    y_rows = _tc_matmul(x2, wt)

    idx2 = (spiral_adj[0].astype(jnp.int32)
            + (jnp.arange(S, dtype=jnp.int32) * N)[None, :]).reshape(-1)
    pad = _NW * _IDXW - N * S
    idx2 = jnp.concatenate([idx2, jnp.zeros((pad,), jnp.int32)])

    out = _sc_gather_reduce(y_rows, idx2, b)
    return out.reshape(B, N, O)


# speedup vs baseline: 2.3485x; 2.3485x over previous
import functools

import jax
import jax.numpy as jnp
from jax import lax
from jax.experimental import pallas as pl
from jax.experimental.pallas import tpu as pltpu
from jax.experimental.pallas import tpu_sc as plsc

_N = 50000
_F = 128
_S = 9
_O = 128

_MM_BLOCK = 400
_MM_GRID = _N // _MM_BLOCK

_NC = 2
_NS = 16
_NW = _NC * _NS
_CPW = 1568
_CPW_LAST = _N - (_NW - 1) * _CPW
_NPAD = _NW * _CPW
_CH = 8
_NCHUNK = _CPW // _CH
_NCHUNK_LAST = _CPW_LAST // _CH
_LANE = 16
_GROUPS = _O // _LANE


def _mm_body(x_ref, wt_ref, *y_refs):
    d = jnp.dot(x_ref[...], wt_ref[...], preferred_element_type=jnp.float32)
    for s in range(_S):
        y_refs[s][...] = d[:, s * _O:(s + 1) * _O]


def _tc_matmul(x2, wt):
    return pl.pallas_call(
        _mm_body,
        grid=(_MM_GRID,),
        in_specs=[
            pl.BlockSpec((_MM_BLOCK, _F), lambda i: (i, 0)),
            pl.BlockSpec((_F, _S * _O), lambda i: (0, 0)),
        ],
        out_specs=[pl.BlockSpec((_MM_BLOCK, _O), lambda i: (i, 0))
                   for _ in range(_S)],
        out_shape=[jax.ShapeDtypeStruct((_N, _O), jnp.float32)
                   for _ in range(_S)],
    )(x2, wt)


@functools.partial(
    pl.kernel,
    out_type=jax.ShapeDtypeStruct((_N, _O), jnp.float32),
    mesh=plsc.VectorSubcoreMesh(core_axis_name="c", subcore_axis_name="s"),
    scratch_types=[
        pltpu.VMEM((_S * _CPW,), jnp.int32),
        pltpu.VMEM((_S * _CH, _O), jnp.float32),
        pltpu.VMEM((_S * _CH, _O), jnp.float32),
        pltpu.VMEM((_CH, _O), jnp.float32),
        pltpu.VMEM((_CH, _O), jnp.float32),
        pltpu.VMEM((_O,), jnp.float32),
        pltpu.SemaphoreType.DMA,
        pltpu.SemaphoreType.DMA,
        pltpu.SemaphoreType.DMA,
        pltpu.SemaphoreType.DMA,
    ],
)
def _sc_gather_reduce(y0, y1, y2, y3, y4, y5, y6, y7, y8,
                      idx_hbm, b_hbm, out_hbm,
                      idx_v, rows0, rows1, outb0, outb1, bias_v,
                      sem0, sem1, semo0, semo1):
    ys = (y0, y1, y2, y3, y4, y5, y6, y7, y8)
    wid = lax.axis_index("s") * _NC + lax.axis_index("c")
    node_base = wid * _CPW
    nchunks = jnp.where(wid == _NW - 1, _NCHUNK_LAST, _NCHUNK)

    for s in range(_S):
        pltpu.sync_copy(idx_hbm.at[pl.ds(s * _NPAD + node_base, _CPW)],
                        idx_v.at[pl.ds(s * _CPW, _CPW)])
    pltpu.sync_copy(b_hbm, bias_v)

    def gathers(g, rows, sem):
        cps = []
        for s in range(_S):
            src = ys[s].at[idx_v.at[pl.ds(s * _CPW + g * _CH, _CH)]]
            cps.append(pltpu.make_async_copy(
                src, rows.at[pl.ds(s * _CH, _CH)], sem))
        return cps

    def start_gathers(g, rows, sem):
        for cp in gathers(g, rows, sem):
            cp.start()

    def wait_gathers(g, rows, sem):
        for cp in gathers(g, rows, sem):
            cp.wait()

    start_gathers(0, rows0, sem0)
    start_gathers(1, rows1, sem1)

    def out_copy(g, outb, semo):
        dst = out_hbm.at[pl.ds(node_base + g * _CH, _CH)]
        return pltpu.make_async_copy(outb, dst, semo)

    def compute(g, rows, outb):
        for n in range(_CH):
            nid = node_base + g * _CH + n
            keep = (nid != _N - 1).astype(jnp.float32)
            for j in range(_GROUPS):
                sl = pl.ds(j * _LANE, _LANE)
                v = rows[n, sl]
                for s in range(1, _S):
                    v = v + rows[s * _CH + n, sl]
                v = v + bias_v[sl]
                v = jnp.where(v > 0.0, v, jnp.exp(v) - 1.0)
                outb[n, sl] = v * keep

    def body(h, carry):
        for parity, (rows, sem, outb, semo) in enumerate((
                (rows0, sem0, outb0, semo0), (rows1, sem1, outb1, semo1))):
            g = 2 * h + parity
            wait_gathers(g, rows, sem)

            @pl.when(g >= 2)
            def _():
                out_copy(g - 2, outb, semo).wait()

            compute(g, rows, outb)
            out_copy(g, outb, semo).start()
            nxt = g + 2

            @pl.when(nxt < nchunks)
            def _():
                start_gathers(nxt, rows, sem)
        return carry

    lax.fori_loop(0, nchunks // 2, body, 0)
    out_copy(nchunks - 2, outb0, semo0).wait()
    out_copy(nchunks - 1, outb1, semo1).wait()


def kernel(x, spiral_adj, W, b):
    B, N, F = x.shape
    S = spiral_adj.shape[-1]
    O = W.shape[0]
    assert (B, N, F, S, O) == (1, _N, _F, _S, _O)

    x2 = x.reshape(N, F)
    wt = jnp.transpose(W.reshape(O, S, F), (2, 1, 0)).reshape(F, S * O)
    ys = _tc_matmul(x2, wt)

    idxT = jnp.pad(spiral_adj[0].astype(jnp.int32).T, ((0, 0), (0, _NPAD - N)))
    idx2 = idxT.reshape(-1)

    out = _sc_gather_reduce(*ys, idx2, b)
    return out.reshape(B, N, O)
